# Initial kernel scaffold; baseline (speedup 1.0000x reference)
import jax
import jax.numpy as jnp
from jax.experimental import pallas as pl


def kernel(node_feats, hyperedge_index, num_hyperedges, W):
    raise NotImplementedError("write your pallas kernel here")


# speedup vs baseline: 5.1621x; 5.1621x over previous
import functools

import jax
import jax.numpy as jnp
from jax import lax
from jax.experimental import pallas as pl
from jax.experimental.pallas import tpu as pltpu
from jax.experimental.pallas import tpu_sc as plsc

N = 10000
H = 10000
D = 256
DH = 128
E_PAD = 163840
H_PAD = 12288
N_PAD = 10240
DUMMY_SEG = 10016

NC, NS = 2, 16
NW = NC * NS
CH2 = E_PAD // NW
CH3 = E_PAD // NS
KB = 128
NB = CH3 // KB
SL = H_PAD // NW
H_ACC = 10112
HROWS2 = H_ACC // NS
ZROWS = H_ACC // NS


def _k1_body(x_ref, w_ref, s_ref):
    x = x_ref[...]
    w = w_ref[...]
    s = jnp.sum(x * w[0][None, None, :], axis=2)
    s_ref[...] = s - jnp.max(s)


def _scores_tc(x3, w):
    return pl.pallas_call(
        _k1_body,
        out_shape=jax.ShapeDtypeStruct((N_PAD // 128, 128), jnp.float32),
    )(x3, w)


_MESH = plsc.VectorSubcoreMesh(core_axis_name="c", subcore_axis_name="s")


@functools.partial(
    pl.kernel,
    out_type=(
        jax.ShapeDtypeStruct((E_PAD,), jnp.float32),
        jax.ShapeDtypeStruct((NW, H_PAD), jnp.float32),
    ),
    mesh=_MESH,
    compiler_params=pltpu.CompilerParams(needs_layout_passes=False),
    scratch_types=[
        pltpu.VMEM((N_PAD,), jnp.float32),
        pltpu.VMEM((CH2,), jnp.int32),
        pltpu.VMEM((CH2,), jnp.int32),
        pltpu.VMEM((CH2,), jnp.float32),
        pltpu.VMEM((H_PAD,), jnp.float32),
    ],
)
def _k2a(scores_hbm, nidx_hbm, hidx_hbm, ex_hbm, part_hbm,
         scores_v, nidx_v, hidx_v, ex_v, seg_v):
    cid = lax.axis_index("c")
    sid = lax.axis_index("s")
    w = sid * NC + cid
    base = w * CH2
    pltpu.sync_copy(scores_hbm, scores_v)
    pltpu.sync_copy(nidx_hbm.at[pl.ds(base, CH2)], nidx_v)
    pltpu.sync_copy(hidx_hbm.at[pl.ds(base, CH2)], hidx_v)

    zero = jnp.zeros((16,), jnp.float32)

    def zbody(i, _):
        seg_v[pl.ds(i * 16, 16)] = zero
        return 0

    lax.fori_loop(0, H_PAD // 16, zbody, 0)

    def body(j, _):
        o = j * 16
        idx = nidx_v[pl.ds(o, 16)]
        s = plsc.load_gather(scores_v, [idx])
        e = jnp.exp(s)
        ex_v[pl.ds(o, 16)] = e
        he = hidx_v[pl.ds(o, 16)]
        plsc.addupdate_scatter(seg_v, [he], e)
        return 0

    lax.fori_loop(0, CH2 // 16, body, 0)
    pltpu.sync_copy(ex_v, ex_hbm.at[pl.ds(base, CH2)])
    pltpu.sync_copy(seg_v, part_hbm.at[w])


@functools.partial(
    pl.kernel,
    out_type=jax.ShapeDtypeStruct((H_PAD,), jnp.float32),
    mesh=_MESH,
    compiler_params=pltpu.CompilerParams(needs_layout_passes=False),
    scratch_types=[
        pltpu.VMEM((NW, SL), jnp.float32),
        pltpu.VMEM((SL,), jnp.float32),
    ],
)
def _k2b(part_hbm, seg_hbm, slab_v, out_v):
    cid = lax.axis_index("c")
    sid = lax.axis_index("s")
    w = sid * NC + cid
    pltpu.sync_copy(part_hbm.at[:, pl.ds(w * SL, SL)], slab_v)

    def body(j, _):
        o = j * 16
        acc = slab_v[0, pl.ds(o, 16)]
        for r in range(1, NW):
            acc = acc + slab_v[r, pl.ds(o, 16)]
        out_v[pl.ds(o, 16)] = acc
        return 0

    lax.fori_loop(0, SL // 16, body, 0)
    pltpu.sync_copy(out_v, seg_hbm.at[pl.ds(w * SL, SL)])


@functools.partial(
    pl.kernel,
    out_type=jax.ShapeDtypeStruct((E_PAD,), jnp.float32),
    mesh=_MESH,
    compiler_params=pltpu.CompilerParams(needs_layout_passes=False),
    scratch_types=[
        pltpu.VMEM((CH2,), jnp.int32),
        pltpu.VMEM((CH2,), jnp.float32),
        pltpu.VMEM((H_PAD,), jnp.float32),
        pltpu.VMEM((CH2,), jnp.float32),
    ],
)
def _k2c(hidx_hbm, ex_hbm, seg_hbm, attn_hbm, hidx_v, ex_v, seg_v, attn_v):
    cid = lax.axis_index("c")
    sid = lax.axis_index("s")
    w = sid * NC + cid
    base = w * CH2
    pltpu.sync_copy(hidx_hbm.at[pl.ds(base, CH2)], hidx_v)
    pltpu.sync_copy(ex_hbm.at[pl.ds(base, CH2)], ex_v)
    pltpu.sync_copy(seg_hbm, seg_v)

    def body(j, _):
        o = j * 16
        he = hidx_v[pl.ds(o, 16)]
        ssum = plsc.load_gather(seg_v, [he])
        attn_v[pl.ds(o, 16)] = ex_v[pl.ds(o, 16)] / ssum
        return 0

    lax.fori_loop(0, CH2 // 16, body, 0)
    pltpu.sync_copy(attn_v, attn_hbm.at[pl.ds(base, CH2)])


@functools.partial(
    pl.kernel,
    out_type=jax.ShapeDtypeStruct((NC, H_ACC, DH), jnp.float32),
    mesh=_MESH,
    compiler_params=pltpu.CompilerParams(needs_layout_passes=False),
    scratch_types=[
        pltpu.VMEM((CH3,), jnp.int32),
        pltpu.VMEM((NB, KB), jnp.int32),
        pltpu.VMEM((KB,), jnp.float32),
        pltpu.VMEM((KB, DH), jnp.float32),
        pltpu.VMEM_SHARED((H_ACC, DH), jnp.float32),
        pltpu.SemaphoreType.DMA,
    ],
)
def _k3(feats_hbm, nidx_hbm, hidx3_hbm, attn_hbm, zeros_hbm, out_hbm,
        nidx_v, hidxb_v, attn_v, rows_v, acc_sh, sem):
    cid = lax.axis_index("c")
    sid = lax.axis_index("s")
    tbase = sid * CH3
    pltpu.sync_copy(nidx_hbm.at[pl.ds(tbase, CH3)], nidx_v)
    pltpu.sync_copy(hidx3_hbm.at[sid], hidxb_v)
    pltpu.sync_copy(zeros_hbm, acc_sh.at[pl.ds(sid * ZROWS, ZROWS)])

    off = (cid * N).astype(jnp.int32)

    def obody(i, _):
        o = i * 16
        nidx_v[pl.ds(o, 16)] = nidx_v[pl.ds(o, 16)] + off
        return 0

    lax.fori_loop(0, CH3 // 16, obody, 0)
    plsc.subcore_barrier()

    def body(b, _):
        ebase = b * KB
        pltpu.sync_copy(attn_hbm.at[pl.ds(tbase + ebase, KB)], attn_v)
        pltpu.async_copy(
            feats_hbm.at[nidx_v.at[pl.ds(ebase, KB)]], rows_v, sem).wait()

        def scale(i, _):
            a = plsc.load_gather(
                attn_v, [jnp.broadcast_to(i, (16,)).astype(jnp.int32)])
            for k in range(DH // 16):
                rows_v[i, pl.ds(k * 16, 16)] = rows_v[i, pl.ds(k * 16, 16)] * a
            return 0

        lax.fori_loop(0, KB, scale, 0)
        pltpu.sync_copy(rows_v, acc_sh.at[hidxb_v.at[b]], add=True)
        return 0

    lax.fori_loop(0, NB, body, 0)
    plsc.subcore_barrier()
    pltpu.sync_copy(acc_sh.at[pl.ds(sid * HROWS2, HROWS2)],
                    out_hbm.at[cid].at[pl.ds(sid * HROWS2, HROWS2)])


def kernel(node_feats, hyperedge_index, num_hyperedges, W):
    del num_hyperedges
    nidx = hyperedge_index[0].astype(jnp.int32)
    hidx = hyperedge_index[1].astype(jnp.int32)
    e = nidx.shape[0]
    pad = E_PAD - e
    nidx = jnp.pad(nidx, (0, pad))
    hidx = jnp.pad(hidx, (0, pad), constant_values=DUMMY_SEG)

    x3 = jnp.pad(node_feats, ((0, N_PAD - N), (0, 0))).reshape(
        N_PAD // 128, 128, D)
    scores = _scores_tc(x3, W).reshape(N_PAD)

    ex, parts = _k2a(scores, nidx, hidx)
    segsum = _k2b(parts)
    attn = _k2c(hidx, ex, segsum)

    feats_cat = jnp.concatenate([node_feats[:, :DH], node_feats[:, DH:]],
                                axis=0)
    hidx3 = hidx.reshape(NS, NB, KB)
    zeros = jnp.zeros((ZROWS, DH), jnp.float32)
    out2 = _k3(feats_cat, nidx, hidx3, attn, zeros)
    return jnp.concatenate([out2[0, :H], out2[1, :H]], axis=1)
